# baseline (speedup 1.0000x reference)
import functools

import jax
import jax.numpy as jnp
from jax import lax
from jax.experimental import pallas as pl
from jax.experimental.pallas import tpu as pltpu
from jax.experimental.pallas import tpu_sc as plsc

N_NODES = 10000
N_EDGES = 320000
D = 128

NC = 2
NS = 16
NW = NC * NS

CHUNK = 128
NQ = N_EDGES // CHUNK
NQT = NQ // NW
NBUF = 3
ROWS_MOST = 640
ROWS_LAST = N_NODES - 15 * ROWS_MOST

_SC_MESH = plsc.VectorSubcoreMesh(
    core_axis_name="c", subcore_axis_name="s", num_cores=NC, num_subcores=NS)


@functools.partial(
    pl.kernel,
    out_type=jax.ShapeDtypeStruct((NC, N_NODES, D), jnp.float32),
    mesh=_SC_MESH,
    scratch_types=[
        pltpu.VMEM_SHARED((N_NODES, D), jnp.float32),
        pltpu.VMEM((NBUF, CHUNK), jnp.int32),
        pltpu.VMEM((NBUF, CHUNK, D), jnp.float32),
        pltpu.SemaphoreType.DMA,
        pltpu.SemaphoreType.DMA,
        pltpu.SemaphoreType.DMA,
        pltpu.SemaphoreType.DMA,
        pltpu.SemaphoreType.DMA,
        pltpu.SemaphoreType.DMA,
        pltpu.SemaphoreType.DMA,
        pltpu.SemaphoreType.DMA,
        pltpu.SemaphoreType.DMA,
    ],
)
def _sc_scatter_add(e_hbm, dst_hbm, out_hbm, acc, idx_v, ebuf,
                    ifill_sem0, ifill_sem1, ifill_sem2,
                    fill_sem0, fill_sem1, fill_sem2,
                    scat_sem0, scat_sem1, scat_sem2):
    c = lax.axis_index("c")
    s = lax.axis_index("s")
    ifill_sems = (ifill_sem0, ifill_sem1, ifill_sem2)
    fill_sems = (fill_sem0, fill_sem1, fill_sem2)
    scat_sems = (scat_sem0, scat_sem1, scat_sem2)

    wid = c * NS + s
    qs = wid * NQT + jnp.maximum(wid - 28, 0)

    def _zrow(i, _):
        def _zcol(j, _):
            ebuf[0, i, pl.ds(j * 16, 16)] = jnp.zeros((16,), jnp.float32)
            return 0
        return lax.fori_loop(0, D // 16, _zcol, 0)
    lax.fori_loop(0, CHUNK, _zrow, 0)

    def _zcp(t, _):
        pltpu.sync_copy(ebuf.at[0],
                        acc.at[pl.ds(s * ROWS_MOST + t * CHUNK, CHUNK)])
        return 0

    @pl.when(s < 15)
    def _():
        lax.fori_loop(0, ROWS_MOST // CHUNK, _zcp, 0)

    @pl.when(s == 15)
    def _():
        lax.fori_loop(0, ROWS_LAST // CHUNK, _zcp, 0)
        pltpu.sync_copy(
            ebuf.at[0, pl.ds(0, ROWS_LAST % CHUNK)],
            acc.at[pl.ds(15 * ROWS_MOST + (ROWS_LAST // CHUNK) * CHUNK,
                         ROWS_LAST % CHUNK)])

    plsc.subcore_barrier()

    def _fill(k, b):
        pltpu.async_copy(dst_hbm.at[qs + k], idx_v.at[b], ifill_sems[b])
        pltpu.async_copy(e_hbm.at[pl.ds((qs + k) * CHUNK, CHUNK)],
                         ebuf.at[b], fill_sems[b])

    def _wait_fill(b):
        pltpu.make_async_copy(dst_hbm.at[qs], idx_v.at[b],
                              ifill_sems[b]).wait()
        pltpu.make_async_copy(e_hbm.at[pl.ds(0, CHUNK)],
                              ebuf.at[b], fill_sems[b]).wait()

    def _drain_scat(b):
        pltpu.make_async_copy(e_hbm.at[pl.ds(0, CHUNK)],
                              ebuf.at[b], scat_sems[b]).wait()

    def _step(k, b, do_drain=True, do_fill=True):
        _wait_fill(b)
        pltpu.async_copy(ebuf.at[b], acc.at[idx_v.at[b]], scat_sems[b],
                         add=True)
        br = (b + 2) % NBUF
        if do_drain:
            _drain_scat(br)
        if do_fill:
            _fill(k + 2, br)

    _fill(0, 0)
    _fill(1, 1)
    _step(0, 0, do_drain=False)
    _step(1, 1)
    _step(2, 2)

    def _triple(t, _):
        k = 3 * t
        _step(k, 0)
        _step(k + 1, 1)
        _step(k + 2, 2)
        return 0
    lax.fori_loop(1, (NQT - 2) // 3, _triple, 0)

    _step(NQT - 3, 0)
    _step(NQT - 2, 1, do_fill=False)
    @pl.when(wid >= 28)
    def _():
        _fill(NQT, 0)

    _step(NQT - 1, 2, do_fill=False)

    @pl.when(wid >= 28)
    def _():
        _step(NQT, 0, do_fill=False)
        _drain_scat(0)

    @pl.when(wid < 28)
    def _():
        _drain_scat(2)
    plsc.subcore_barrier()

    @pl.when(s < 15)
    def _():
        pltpu.sync_copy(acc.at[pl.ds(s * ROWS_MOST, ROWS_MOST)],
                        out_hbm.at[c, pl.ds(s * ROWS_MOST, ROWS_MOST)])

    @pl.when(s == 15)
    def _():
        pltpu.sync_copy(acc.at[pl.ds(15 * ROWS_MOST, ROWS_LAST)],
                        out_hbm.at[c, pl.ds(15 * ROWS_MOST, ROWS_LAST)])


_ROWS_BLK = 1000


def _matmul_t(x, w):
    return lax.dot_general(x, w, (((1,), (1,)), ((), ())),
                           preferred_element_type=jnp.float32)


def _mlp_body(p0_ref, p1_ref, v_ref, w1_ref, b1_ref, w2_ref, b2_ref, o_ref):
    x = p0_ref[...] + p1_ref[...]
    h = _matmul_t(x, w1_ref[...]) + b1_ref[...]
    sp = jnp.maximum(h, 0.0) + jnp.log1p(jnp.exp(-jnp.abs(h)))
    sp = sp - jnp.log(jnp.float32(2.0))
    o_ref[...] = v_ref[...] + b2_ref[...] + _matmul_t(sp, w2_ref[...])


def _mlp(p0, p1, v, W1, b1, W2, b2):
    grid = (N_NODES // _ROWS_BLK,)
    row_spec = pl.BlockSpec((_ROWS_BLK, D), lambda i: (i, 0))
    w_spec = pl.BlockSpec((D, D), lambda i: (0, 0))
    b_spec = pl.BlockSpec((1, D), lambda i: (0, 0))
    return pl.pallas_call(
        _mlp_body,
        grid=grid,
        in_specs=[row_spec, row_spec, row_spec, w_spec, b_spec, w_spec, b_spec],
        out_specs=row_spec,
        out_shape=jax.ShapeDtypeStruct((N_NODES, D), jnp.float32),
    )(p0, p1, v, W1, b1, W2, b2)


def kernel(v, e, edge_index, W1, b1, W2, b2):
    dst = edge_index[1].reshape(NQ, CHUNK)
    partials = _sc_scatter_add(e, dst)
    return _mlp(partials[0], partials[1], v,
                W1, b1.reshape(1, D), W2, b2.reshape(1, D))


# speedup vs baseline: 1.0695x; 1.0695x over previous
import functools

import jax
import jax.numpy as jnp
from jax import lax
from jax.experimental import pallas as pl
from jax.experimental.pallas import tpu as pltpu
from jax.experimental.pallas import tpu_sc as plsc

N_NODES = 10000
N_EDGES = 320000
D = 128

NC = 2
NS = 16
NW = NC * NS

EPT = N_EDGES // NW
CHUNK = 80
NCHUNK = EPT // CHUNK
NBUF = 3
ROWS_MOST = 640
ROWS_LAST = N_NODES - 15 * ROWS_MOST
ZROWS = 16

_SC_MESH = plsc.VectorSubcoreMesh(
    core_axis_name="c", subcore_axis_name="s", num_cores=NC, num_subcores=NS)


@functools.partial(
    pl.kernel,
    out_type=jax.ShapeDtypeStruct((NC, N_NODES, D), jnp.float32),
    mesh=_SC_MESH,
    scratch_types=[
        pltpu.VMEM_SHARED((N_NODES, D), jnp.float32),
        pltpu.VMEM((NCHUNK, CHUNK), jnp.int32),
        pltpu.VMEM((NBUF, CHUNK, D), jnp.float32),
        pltpu.VMEM((ZROWS, D), jnp.float32),
        pltpu.SemaphoreType.DMA,
        pltpu.SemaphoreType.DMA,
        pltpu.SemaphoreType.DMA,
        pltpu.SemaphoreType.DMA,
        pltpu.SemaphoreType.DMA,
        pltpu.SemaphoreType.DMA,
    ],
)
def _sc_scatter_add(e_hbm, dst_hbm, out_hbm, acc, idx_v, ebuf, zbuf,
                    fill_sem0, fill_sem1, fill_sem2,
                    scat_sem0, scat_sem1, scat_sem2):
    c = lax.axis_index("c")
    s = lax.axis_index("s")
    fill_sems = (fill_sem0, fill_sem1, fill_sem2)
    scat_sems = (scat_sem0, scat_sem1, scat_sem2)

    base = (c * NS + s) * EPT
    wid = c * NS + s

    def _fill(j, b):
        pltpu.async_copy(e_hbm.at[pl.ds(base + j * CHUNK, CHUNK)],
                         ebuf.at[b], fill_sems[b])

    pltpu.async_copy(dst_hbm.at[wid], idx_v, scat_sem0)
    _fill(0, 0)
    _fill(1, 1)

    def _zrow(r, _):
        def _zcol(j, _):
            zbuf[r, pl.ds(j * 16, 16)] = jnp.zeros((16,), jnp.float32)
            return 0
        return lax.fori_loop(0, D // 16, _zcol, 0)
    lax.fori_loop(0, ZROWS, _zrow, 0)

    def _zcp(t, _):
        pltpu.sync_copy(zbuf, acc.at[pl.ds(s * ROWS_MOST + t * ZROWS, ZROWS)])
        return 0

    @pl.when(s < 15)
    def _():
        lax.fori_loop(0, ROWS_MOST // ZROWS, _zcp, 0)

    @pl.when(s == 15)
    def _():
        lax.fori_loop(0, ROWS_LAST // ZROWS, _zcp, 0)

    pltpu.make_async_copy(dst_hbm.at[wid], idx_v, scat_sem0).wait()
    plsc.subcore_barrier()

    def _wait_fill(b):
        pltpu.make_async_copy(e_hbm.at[pl.ds(base, CHUNK)],
                              ebuf.at[b], fill_sems[b]).wait()

    def _drain_scat(b):
        pltpu.make_async_copy(e_hbm.at[pl.ds(base, CHUNK)],
                              ebuf.at[b], scat_sems[b]).wait()

    def _step(j, b, do_drain=True, do_fill=True):
        _wait_fill(b)
        pltpu.async_copy(ebuf.at[b], acc.at[idx_v.at[j]], scat_sems[b],
                         add=True)
        br = (b + 2) % NBUF
        if do_drain:
            _drain_scat(br)
        if do_fill:
            _fill(j + 2, br)

    _step(0, 0, do_drain=False)
    _step(1, 1)
    _step(2, 2)

    def _triple(t, _):
        j = 3 * t
        _step(j, 0)
        _step(j + 1, 1)
        _step(j + 2, 2)
        return 0
    lax.fori_loop(1, (NCHUNK - 2) // 3, _triple, 0)
    _step(NCHUNK - 2, 0, do_fill=False)
    _step(NCHUNK - 1, 1, do_fill=False)
    _drain_scat(1)
    plsc.subcore_barrier()

    @pl.when(s < 15)
    def _():
        pltpu.sync_copy(acc.at[pl.ds(s * ROWS_MOST, ROWS_MOST)],
                        out_hbm.at[c, pl.ds(s * ROWS_MOST, ROWS_MOST)])

    @pl.when(s == 15)
    def _():
        pltpu.sync_copy(acc.at[pl.ds(15 * ROWS_MOST, ROWS_LAST)],
                        out_hbm.at[c, pl.ds(15 * ROWS_MOST, ROWS_LAST)])


_ROWS_BLK = 2000


def _matmul_t(x, w):
    return lax.dot_general(x, w, (((1,), (1,)), ((), ())),
                           preferred_element_type=jnp.float32)


def _mlp_body(p0_ref, p1_ref, v_ref, w1_ref, b1_ref, w2_ref, b2_ref, o_ref):
    x = p0_ref[...] + p1_ref[...]
    h = _matmul_t(x, w1_ref[...]) + b1_ref[...]
    sp = jnp.maximum(h, 0.0) + jnp.log1p(jnp.exp(-jnp.abs(h)))
    sp = sp - jnp.log(jnp.float32(2.0))
    o_ref[...] = v_ref[...] + b2_ref[...] + _matmul_t(sp, w2_ref[...])


def _mlp(p0, p1, v, W1T, b1, W2T, b2):
    grid = (N_NODES // _ROWS_BLK,)
    row_spec = pl.BlockSpec((_ROWS_BLK, D), lambda i: (i, 0))
    w_spec = pl.BlockSpec((D, D), lambda i: (0, 0))
    b_spec = pl.BlockSpec((1, D), lambda i: (0, 0))
    return pl.pallas_call(
        _mlp_body,
        grid=grid,
        in_specs=[row_spec, row_spec, row_spec, w_spec, b_spec, w_spec, b_spec],
        out_specs=row_spec,
        out_shape=jax.ShapeDtypeStruct((N_NODES, D), jnp.float32),
    )(p0, p1, v, W1T, b1, W2T, b2)


def kernel(v, e, edge_index, W1, b1, W2, b2):
    dst = edge_index[1].reshape(NW, NCHUNK, CHUNK)
    partials = _sc_scatter_add(e, dst)
    return _mlp(partials[0], partials[1], v,
                W1, b1.reshape(1, D), W2, b2.reshape(1, D))


# speedup vs baseline: 1.0804x; 1.0102x over previous
import functools

import jax
import jax.numpy as jnp
from jax import lax
from jax.experimental import pallas as pl
from jax.experimental.pallas import tpu as pltpu
from jax.experimental.pallas import tpu_sc as plsc

N_NODES = 10000
N_EDGES = 320000
D = 128

NC = 2
NS = 16
NW = NC * NS

EPT = N_EDGES // NW
CHUNK = 80
NCHUNK = EPT // CHUNK
NBUF = 3
ROWS_MOST = 640
ROWS_LAST = N_NODES - 15 * ROWS_MOST
ZROWS = 16

_SC_MESH = plsc.VectorSubcoreMesh(
    core_axis_name="c", subcore_axis_name="s", num_cores=NC, num_subcores=NS)


@functools.partial(
    pl.kernel,
    out_type=jax.ShapeDtypeStruct((NC, N_NODES, D), jnp.float32),
    mesh=_SC_MESH,
    scratch_types=[
        pltpu.VMEM_SHARED((N_NODES, D), jnp.float32),
        pltpu.VMEM((NCHUNK, CHUNK), jnp.int32),
        pltpu.VMEM((NBUF, CHUNK, D), jnp.float32),
        pltpu.VMEM((ZROWS, D), jnp.float32),
        pltpu.SemaphoreType.DMA,
        pltpu.SemaphoreType.DMA,
        pltpu.SemaphoreType.DMA,
        pltpu.SemaphoreType.DMA,
        pltpu.SemaphoreType.DMA,
        pltpu.SemaphoreType.DMA,
    ],
)
def _sc_scatter_add(e_hbm, dst_hbm, out_hbm, acc, idx_v, ebuf, zbuf,
                    fill_sem0, fill_sem1, fill_sem2,
                    scat_sem0, scat_sem1, scat_sem2):
    c = lax.axis_index("c")
    s = lax.axis_index("s")
    fill_sems = (fill_sem0, fill_sem1, fill_sem2)
    scat_sems = (scat_sem0, scat_sem1, scat_sem2)

    base = (c * NS + s) * EPT
    wid = c * NS + s

    def _fill(j, b):
        pltpu.async_copy(e_hbm.at[pl.ds(base + j * CHUNK, CHUNK)],
                         ebuf.at[b], fill_sems[b])

    pltpu.async_copy(dst_hbm.at[wid], idx_v, scat_sem0)
    _fill(0, 0)
    _fill(1, 1)

    def _zrow(r, _):
        def _zcol(j, _):
            zbuf[r, pl.ds(j * 16, 16)] = jnp.zeros((16,), jnp.float32)
            return 0
        return lax.fori_loop(0, D // 16, _zcol, 0)
    lax.fori_loop(0, ZROWS, _zrow, 0)

    def _zcp(t, _):
        pltpu.sync_copy(zbuf, acc.at[pl.ds(s * ROWS_MOST + t * ZROWS, ZROWS)])
        return 0

    @pl.when(s < 15)
    def _():
        lax.fori_loop(0, ROWS_MOST // ZROWS, _zcp, 0)

    @pl.when(s == 15)
    def _():
        lax.fori_loop(0, ROWS_LAST // ZROWS, _zcp, 0)

    pltpu.make_async_copy(dst_hbm.at[wid], idx_v, scat_sem0).wait()
    plsc.subcore_barrier()

    def _wait_fill(b):
        pltpu.make_async_copy(e_hbm.at[pl.ds(base, CHUNK)],
                              ebuf.at[b], fill_sems[b]).wait()

    def _drain_scat(b):
        pltpu.make_async_copy(e_hbm.at[pl.ds(base, CHUNK)],
                              ebuf.at[b], scat_sems[b]).wait()

    def _step(j, b, do_drain=True, do_fill=True):
        _wait_fill(b)
        pltpu.async_copy(ebuf.at[b], acc.at[idx_v.at[j]], scat_sems[b],
                         add=True)
        br = (b + 2) % NBUF
        if do_drain:
            _drain_scat(br)
        if do_fill:
            _fill(j + 2, br)

    _step(0, 0, do_drain=False)
    _step(1, 1)
    _step(2, 2)

    def _triple(t, _):
        j = 3 * t
        _step(j, 0)
        _step(j + 1, 1)
        _step(j + 2, 2)
        return 0
    lax.fori_loop(1, (NCHUNK - 2) // 3, _triple, 0)
    _step(NCHUNK - 2, 0, do_fill=False)
    _step(NCHUNK - 1, 1, do_fill=False)
    _drain_scat(1)
    plsc.subcore_barrier()

    @pl.when(s < 15)
    def _():
        pltpu.sync_copy(acc.at[pl.ds(s * ROWS_MOST, ROWS_MOST)],
                        out_hbm.at[c, pl.ds(s * ROWS_MOST, ROWS_MOST)])

    @pl.when(s == 15)
    def _():
        pltpu.sync_copy(acc.at[pl.ds(15 * ROWS_MOST, ROWS_LAST)],
                        out_hbm.at[c, pl.ds(15 * ROWS_MOST, ROWS_LAST)])


_ROWS_BLK = 5000


def _matmul_t(x, w):
    return lax.dot_general(x, w, (((1,), (1,)), ((), ())),
                           preferred_element_type=jnp.float32)


def _mlp_body(p0_ref, p1_ref, v_ref, w1_ref, b1_ref, w2_ref, b2_ref, o_ref):
    x = p0_ref[...] + p1_ref[...]
    h = _matmul_t(x, w1_ref[...]) + b1_ref[...]
    sp = jnp.maximum(h, 0.0) + jnp.log1p(jnp.exp(-jnp.abs(h)))
    sp = sp - jnp.log(jnp.float32(2.0))
    o_ref[...] = v_ref[...] + b2_ref[...] + _matmul_t(sp, w2_ref[...])


def _mlp(p0, p1, v, W1T, b1, W2T, b2):
    grid = (N_NODES // _ROWS_BLK,)
    row_spec = pl.BlockSpec((_ROWS_BLK, D), lambda i: (i, 0))
    w_spec = pl.BlockSpec((D, D), lambda i: (0, 0))
    b_spec = pl.BlockSpec((1, D), lambda i: (0, 0))
    return pl.pallas_call(
        _mlp_body,
        grid=grid,
        in_specs=[row_spec, row_spec, row_spec, w_spec, b_spec, w_spec, b_spec],
        out_specs=row_spec,
        out_shape=jax.ShapeDtypeStruct((N_NODES, D), jnp.float32),
    )(p0, p1, v, W1T, b1, W2T, b2)


def kernel(v, e, edge_index, W1, b1, W2, b2):
    dst = edge_index[1].reshape(NW, NCHUNK, CHUNK)
    partials = _sc_scatter_add(e, dst)
    return _mlp(partials[0], partials[1], v,
                W1, b1.reshape(1, D), W2, b2.reshape(1, D))
